# Initial kernel scaffold; baseline (speedup 1.0000x reference)
import functools

import jax
import jax.numpy as jnp
from jax import lax
from jax.experimental import pallas as pl
from jax.experimental.pallas import tpu as pltpu
from jax.experimental.pallas import tpu_sc as plsc

NC = 2
NS = 16
NW = NC * NS

IB = 128
ROWS_PER_CHUNK = 5
CHUNK = IB * ROWS_PER_CHUNK


def _sc_body(ids_ref, base_ref, prior_ref, gate_ref, out_ref,
             idx_v, base_v, prior_v, gate_v, sem, *, rows_per_worker, d):
    wid = lax.axis_index("s") * NC + lax.axis_index("c")
    row0 = wid * rows_per_worker
    n_chunks = rows_per_worker // ROWS_PER_CHUNK

    pltpu.sync_copy(ids_ref.at[pl.ds(row0, rows_per_worker)], idx_v)

    for c in range(n_chunks):
        copies = []
        for j in range(ROWS_PER_CHUNK):
            r = c * ROWS_PER_CHUNK + j
            idx_row = idx_v.at[r]
            dst = pl.ds(j * IB, IB)
            copies.append(pltpu.async_copy(base_ref.at[idx_row], base_v.at[dst], sem))
            copies.append(pltpu.async_copy(prior_ref.at[idx_row], prior_v.at[dst], sem))
            copies.append(pltpu.async_copy(gate_ref.at[idx_row], gate_v.at[dst], sem))
        for cp in copies:
            cp.wait()

        def combine(row, _):
            g = plsc.load_gather(gate_v, [jnp.full((16,), row, jnp.int32)])
            w = 1.0 / (1.0 + jnp.exp(-g))
            for k in range(d // 16):
                sl = pl.ds(k * 16, 16)
                prior_v[row, sl] = base_v[row, sl] + w * prior_v[row, sl]
            return 0

        lax.fori_loop(0, CHUNK, combine, 0)

        out0 = (row0 + c * ROWS_PER_CHUNK) * IB
        pltpu.sync_copy(prior_v, out_ref.at[pl.ds(out0, CHUNK)])


def kernel(input_ids, base_weight, prior_matrix, gate_logits):
    b, l = input_ids.shape
    v, d = base_weight.shape
    n = b * l
    assert n % (NW * IB) == 0 and d % 16 == 0
    rows_per_worker = n // (NW * IB)
    assert rows_per_worker % ROWS_PER_CHUNK == 0

    ids2 = input_ids.reshape(n // IB, IB)

    mesh = plsc.VectorSubcoreMesh(core_axis_name="c", subcore_axis_name="s")
    body = functools.partial(_sc_body, rows_per_worker=rows_per_worker, d=d)
    call = pl.kernel(
        body,
        mesh=mesh,
        out_type=jax.ShapeDtypeStruct((n, d), jnp.float32),
        scratch_types=[
            pltpu.VMEM((rows_per_worker, IB), jnp.int32),
            pltpu.VMEM((CHUNK, d), jnp.float32),
            pltpu.VMEM((CHUNK, d), jnp.float32),
            pltpu.VMEM((CHUNK,), jnp.float32),
            pltpu.SemaphoreType.DMA,
        ],
    )
    out = call(ids2, base_weight, prior_matrix, gate_logits)
    return out.reshape(b, l, d)


# speedup vs baseline: 10.0495x; 10.0495x over previous
import functools

import jax
import jax.numpy as jnp
from jax import lax
from jax.experimental import pallas as pl
from jax.experimental.pallas import tpu as pltpu
from jax.experimental.pallas import tpu_sc as plsc

NC = 2
NS = 16
NW = NC * NS

IB = 128
ROWS_PER_CHUNK = 5
CHUNK = IB * ROWS_PER_CHUNK


def _sc_body(ids_ref, base_ref, prior_ref, gate_ref, out_ref,
             idx_v, base_v, prior_v, gate_v, sem, *, rows_per_worker, d):
    wid = lax.axis_index("s") * NC + lax.axis_index("c")
    row0 = wid * rows_per_worker
    n_chunks = rows_per_worker // ROWS_PER_CHUNK

    pltpu.sync_copy(ids_ref.at[wid], idx_v)

    for c in range(n_chunks):
        copies = []
        for j in range(ROWS_PER_CHUNK):
            r = c * ROWS_PER_CHUNK + j
            idx_row = idx_v.at[r]
            dst = pl.ds(j * IB, IB)
            copies.append(pltpu.async_copy(base_ref.at[idx_row], base_v.at[dst], sem))
            copies.append(pltpu.async_copy(prior_ref.at[idx_row], prior_v.at[dst], sem))
            copies.append(pltpu.async_copy(gate_ref.at[idx_row], gate_v.at[dst], sem))
        for cp in copies:
            cp.wait()

        dnums = lax.GatherDimensionNumbers(
            offset_dims=(), collapsed_slice_dims=(0,), start_index_map=(0,))

        def combine(grp, _):
            g16 = gate_v[pl.ds(grp * 16, 16)]
            w16 = 1.0 / (1.0 + jnp.exp(-g16))
            for j in range(16):
                row = grp * 16 + j
                w = lax.gather(
                    w16, jnp.full((16, 1), j, jnp.int32), dnums,
                    slice_sizes=(1,),
                    mode=lax.GatherScatterMode.PROMISE_IN_BOUNDS)
                for k in range(d // 16):
                    sl = pl.ds(k * 16, 16)
                    prior_v[row, sl] = base_v[row, sl] + w * prior_v[row, sl]
            return 0

        lax.fori_loop(0, CHUNK // 16, combine, 0)

        out0 = (row0 + c * ROWS_PER_CHUNK) * IB
        pltpu.sync_copy(prior_v, out_ref.at[pl.ds(out0, CHUNK)])


def kernel(input_ids, base_weight, prior_matrix, gate_logits):
    b, l = input_ids.shape
    v, d = base_weight.shape
    n = b * l
    assert n % (NW * IB) == 0 and d % 16 == 0
    rows_per_worker = n // (NW * IB)
    assert rows_per_worker % ROWS_PER_CHUNK == 0

    ids2 = input_ids.reshape(NW, rows_per_worker, IB)

    mesh = plsc.VectorSubcoreMesh(core_axis_name="c", subcore_axis_name="s")
    body = functools.partial(_sc_body, rows_per_worker=rows_per_worker, d=d)
    call = pl.kernel(
        body,
        mesh=mesh,
        compiler_params=pltpu.CompilerParams(use_tc_tiling_on_sc=False),
        out_type=jax.ShapeDtypeStruct((n, d), jnp.float32),
        scratch_types=[
            pltpu.VMEM((rows_per_worker, IB), jnp.int32),
            pltpu.VMEM((CHUNK, d), jnp.float32),
            pltpu.VMEM((CHUNK, d), jnp.float32),
            pltpu.VMEM((CHUNK,), jnp.float32),
            pltpu.SemaphoreType.DMA,
        ],
    )
    out = call(ids2, base_weight, prior_matrix, gate_logits)
    return out.reshape(b, l, d)
